# baseline (speedup 1.0000x reference)
import jax
import jax.numpy as jnp
from jax import lax
from jax.experimental import pallas as pl
from jax.experimental.pallas import tpu as pltpu


def _make_block_kernel(Bblk, Hh, Wh, D, Kp):
    HW = Hh * Wh
    BH = Bblk * HW
    f32 = jnp.float32

    def _body(p1_ref, xs_ref, w1_ref, alpha_ref, w2_ref, bn2b_ref,
              wsc_ref, bnscb_ref, wfc1_ref, wfc2_ref, o_ref):
        def dot(a, b):
            return jnp.dot(a, b, preferred_element_type=f32)

        pv = p1_ref[...].reshape(Bblk, Hh, 2, Wh, 2 * Kp)

        def phase(ry, rx):
            patch = pv[:, :, ry, :, rx * Kp:(rx + 1) * Kp].reshape(BH, Kp)
            y = dot(patch, w1_ref[...])
            return jnp.where(y >= 0.0, y, y * alpha_ref[...])

        ph0 = phase(0, 0)
        ph1 = phase(0, 1)
        ph2 = phase(1, 0)
        ph3 = phase(1, 1)

        d00 = (dot(ph0, w2_ref[4]) + dot(ph1, w2_ref[5])
               + dot(ph2, w2_ref[7]) + dot(ph3, w2_ref[8]))
        d01 = dot(ph1, w2_ref[3]) + dot(ph3, w2_ref[6])
        d10 = dot(ph2, w2_ref[1]) + dot(ph3, w2_ref[2])
        d11 = dot(ph3, w2_ref[0])

        def shift_n(v):
            r = v.reshape(Bblk * Hh, Wh, D)
            r = jnp.concatenate(
                [jnp.zeros((Bblk * Hh, 1, D), f32), r[:, :Wh - 1, :]], axis=1)
            return r.reshape(BH, D)

        def shift_m(v):
            r = v.reshape(Bblk, Hh, Wh * D)
            r = jnp.concatenate(
                [jnp.zeros((Bblk, 1, Wh * D), f32), r[:, :Hh - 1, :]], axis=1)
            return r.reshape(BH, D)

        y2 = d00 + shift_n(d01) + shift_m(d10) + shift_m(shift_n(d11))
        y2 = y2 + bn2b_ref[...]

        short = dot(xs_ref[...], wsc_ref[...]) + bnscb_ref[...]

        y2b = y2.reshape(Bblk, HW, D)
        pooled = jnp.mean(y2b, axis=1)
        h = jnp.maximum(dot(pooled, wfc1_ref[...]), 0.0)
        s = jax.nn.sigmoid(dot(h, wfc2_ref[...]))

        out = y2b * s[:, None, :] + short.reshape(Bblk, HW, D)
        o_ref[...] = out.reshape(BH, D)

    return _body


def _bottleneck(patch1, xs, w1, alpha, w2, bn2b, wsc, bnscb, wfc1, wfc2,
                *, Bblk, Hh, Wh):
    G = patch1.shape[0]
    Kp = patch1.shape[-1] // 2
    Csp = xs.shape[-1]
    D = w2.shape[-1]
    Dmid = wfc1.shape[-1]
    BH = Bblk * Hh * Wh
    body = _make_block_kernel(Bblk, Hh, Wh, D, Kp)

    def const(shape):
        return pl.BlockSpec(shape, lambda g: (0,) * len(shape))

    return pl.pallas_call(
        body,
        out_shape=jax.ShapeDtypeStruct((G, BH, D), jnp.float32),
        grid=(G,),
        in_specs=[
            pl.BlockSpec((None, Bblk * 2 * Hh, Wh, 2 * Kp),
                         lambda g: (g, 0, 0, 0)),
            pl.BlockSpec((None, BH, Csp), lambda g: (g, 0, 0)),
            const((Kp, D)),
            const((1, D)),
            const((9, D, D)),
            const((1, D)),
            const((Csp, D)),
            const((1, D)),
            const((D, Dmid)),
            const((Dmid, D)),
        ],
        out_specs=pl.BlockSpec((None, BH, D), lambda g: (g, 0, 0)),
        compiler_params=pltpu.CompilerParams(dimension_semantics=("parallel",)),
    )(patch1, xs, w1, alpha, w2, bn2b, wsc, bnscb, wfc1, wfc2)


def _make_head_kernel(paired):
    f32 = jnp.float32

    if paired:
        def _body(a_ref, b_ref, wo_ref, wa_ref, wb_ref, o_ref, acca_ref, accb_ref):
            k = pl.program_id(1)

            @pl.when(k == 0)
            def _():
                acca_ref[...] = jnp.zeros_like(acca_ref)
                accb_ref[...] = jnp.zeros_like(accb_ref)

            acca_ref[...] += jnp.dot(a_ref[...], wo_ref[...],
                                     preferred_element_type=f32)
            accb_ref[...] += jnp.dot(b_ref[...], wo_ref[...],
                                     preferred_element_type=f32)

            @pl.when(k == pl.num_programs(1) - 1)
            def _():
                ha = jnp.maximum(jnp.dot(acca_ref[...], wa_ref[...],
                                         preferred_element_type=f32), 0.0)
                hb = jnp.maximum(jnp.dot(accb_ref[...], wa_ref[...],
                                         preferred_element_type=f32), 0.0)
                s = ha + hb
                z = jnp.dot(s, wb_ref[...], preferred_element_type=f32)
                nrm2 = jnp.sum(z * z, axis=1, keepdims=True)
                o_ref[...] = z * lax.rsqrt(jnp.maximum(nrm2, 1e-24))
        return _body

    def _body1(a_ref, wo_ref, wa_ref, wb_ref, o_ref, acca_ref):
        k = pl.program_id(1)

        @pl.when(k == 0)
        def _():
            acca_ref[...] = jnp.zeros_like(acca_ref)

        acca_ref[...] += jnp.dot(a_ref[...], wo_ref[...],
                                 preferred_element_type=f32)

        @pl.when(k == pl.num_programs(1) - 1)
        def _():
            h = jnp.maximum(jnp.dot(acca_ref[...], wa_ref[...],
                                    preferred_element_type=f32), 0.0)
            z = jnp.dot(h, wb_ref[...], preferred_element_type=f32)
            nrm2 = jnp.sum(z * z, axis=1, keepdims=True)
            o_ref[...] = z * lax.rsqrt(jnp.maximum(nrm2, 1e-24))
    return _body1


def _head(feat_flat, w_out, w_a, w_b, *, paired, tk=4096):
    B, F = feat_flat.shape
    E = w_out.shape[-1]
    if F % tk:
        tk = F
    nk = F // tk
    Bout = B // 2 if paired else B
    S = 2 if Bout % 2 == 0 else 1
    R = Bout // S

    in_specs = [pl.BlockSpec((R, tk), lambda c, k: (c, k))]
    scratch = [pltpu.VMEM((R, E), jnp.float32)]
    if paired:
        in_specs.append(pl.BlockSpec((R, tk), lambda c, k: (c + S, k)))
        scratch.append(pltpu.VMEM((R, E), jnp.float32))
    in_specs += [
        pl.BlockSpec((tk, E), lambda c, k: (k, 0)),
        pl.BlockSpec((E, E), lambda c, k: (0, 0)),
        pl.BlockSpec((E, E), lambda c, k: (0, 0)),
    ]
    operands = ([feat_flat, feat_flat] if paired else [feat_flat])
    operands += [w_out, w_a, w_b]

    return pl.pallas_call(
        _make_head_kernel(paired),
        out_shape=jax.ShapeDtypeStruct((Bout, E), jnp.float32),
        grid=(S, nk),
        in_specs=in_specs,
        out_specs=pl.BlockSpec((R, E), lambda c, k: (c, 0)),
        scratch_shapes=scratch,
        compiler_params=pltpu.CompilerParams(
            dimension_semantics=("parallel", "arbitrary")),
    )(*operands)


def _pick_bblk(B):
    for cand in (4, 2):
        if B % cand == 0 and B // cand >= 2:
            return cand
    return 1


def kernel(x_nchw, w_conv1, w_conv2, w_sc, w_fc1, w_fc2, prelu_alpha,
           bn1_scale, bn1_shift, bn2_scale, bn2_shift, bnsc_scale, bnsc_shift,
           w_out, w_drl1, w_drl2):
    Cin = bn1_scale.shape[0]
    D = bn2_scale.shape[0]
    teacher = x_nchw.shape[1] == 2 * Cin

    if teacher:
        x_nchw = x_nchw[:, Cin:, :, :]
        x_nchw = jnp.concatenate([x_nchw, x_nchw[:, :, :, ::-1]], axis=0)
    x = jnp.transpose(x_nchw, (0, 2, 3, 1)).astype(jnp.float32)
    B, H, W, _ = x.shape
    Hh, Wh = H // 2, W // 2
    HW = Hh * Wh
    Bblk = _pick_bblk(B)
    G = B // Bblk
    BH = Bblk * HW
    K9 = 9 * Cin
    Kp = max(64, ((K9 + 63) // 64) * 64)
    Csp = 8

    xbn = x * bn1_scale + bn1_shift
    xp = jnp.pad(xbn, ((0, 0), (1, 1), (1, 1), (0, 0)))
    taps = [xp[:, dy:dy + H, dx:dx + W, :] for dy in range(3) for dx in range(3)]
    kpad = jnp.zeros((B, H, W, Kp - K9), jnp.float32)
    patch1 = jnp.concatenate(taps + [kpad], axis=-1)
    patch1 = patch1.reshape(G, Bblk * H, Wh, 2 * Kp)

    xs = x[:, ::2, ::2, :].reshape(G, BH, Cin)
    xs = jnp.pad(xs, ((0, 0), (0, 0), (0, Csp - Cin)))

    w1 = jnp.pad(w_conv1.reshape(K9, D), ((0, Kp - K9), (0, 0)))
    w2 = (w_conv2 * bn2_scale[None, None, None, :]).reshape(9, D, D)
    wsc = jnp.pad(w_sc * bnsc_scale[None, :], ((0, Csp - Cin), (0, 0)))
    alpha = prelu_alpha.reshape(1, D)
    bn2b = bn2_shift.reshape(1, D)
    bnscb = bnsc_shift.reshape(1, D)

    feat = _bottleneck(patch1, xs, w1, alpha, w2, bn2b, wsc, bnscb,
                       w_fc1, w_fc2, Bblk=Bblk, Hh=Hh, Wh=Wh)
    feat_flat = feat.reshape(B, HW * D)

    return _head(feat_flat, w_out, w_drl1, w_drl2, paired=teacher)


# speedup vs baseline: 5.2362x; 5.2362x over previous
import jax
import jax.numpy as jnp
from jax import lax
from jax.experimental import pallas as pl
from jax.experimental.pallas import tpu as pltpu


def _make_block_kernel(Bblk, Hh, Wh, D, L):
    HW = Hh * Wh
    BH = Bblk * HW
    BHh = Bblk * Hh
    f32 = jnp.float32

    def _body(xpl_ref, wt_ref, alpha_ref, w2_ref, bn2b_ref,
              wsc_ref, bnscb_ref, wfc1_ref, wfc2_ref, o_ref):
        def dot(a, b):
            return jnp.dot(a, b, preferred_element_type=f32)

        xv = xpl_ref[...].reshape(Bblk, Hh + 1, 2, L)
        xe = xv[:, :, 0, :]
        xo = xv[:, :, 1, :]

        def rows(src, off):
            return src[:, off:off + Hh, :].reshape(BHh, L)

        def phase(ry, rx):
            if ry == 0:
                srcs = (rows(xe, 0), rows(xo, 0), rows(xe, 1))
            else:
                srcs = (rows(xo, 0), rows(xe, 1), rows(xo, 1))
            y = (dot(srcs[0], wt_ref[0 * 2 + rx])
                 + dot(srcs[1], wt_ref[1 * 2 + rx])
                 + dot(srcs[2], wt_ref[2 * 2 + rx]))
            return jnp.where(y >= 0.0, y, y * alpha_ref[...])

        y00 = phase(0, 0)
        y01 = phase(0, 1)
        y10 = phase(1, 0)
        y11 = phase(1, 1)

        shf = dot(rows(xo, 0), wsc_ref[...])

        def to_rows(yf):
            pieces = [yf[j * Hh:(j + 1) * Hh, n * D:(n + 1) * D]
                      for j in range(Bblk) for n in range(Wh)]
            return jnp.concatenate(pieces, axis=0)

        ph0 = to_rows(y00)
        ph1 = to_rows(y01)
        ph2 = to_rows(y10)
        ph3 = to_rows(y11)
        short = to_rows(shf) + bnscb_ref[...]

        d00 = (dot(ph0, w2_ref[4]) + dot(ph1, w2_ref[5])
               + dot(ph2, w2_ref[7]) + dot(ph3, w2_ref[8]))
        d01 = dot(ph1, w2_ref[3]) + dot(ph3, w2_ref[6])
        d10 = dot(ph2, w2_ref[1]) + dot(ph3, w2_ref[2])
        d11 = dot(ph3, w2_ref[0])

        def shift_n(v):
            r = v.reshape(Bblk, Wh, Hh * D)
            r = jnp.concatenate(
                [jnp.zeros((Bblk, 1, Hh * D), f32), r[:, :Wh - 1, :]], axis=1)
            return r.reshape(BH, D)

        def shift_m(v):
            r = v.reshape(Bblk * Wh, Hh, D)
            r = jnp.concatenate(
                [jnp.zeros((Bblk * Wh, 1, D), f32), r[:, :Hh - 1, :]], axis=1)
            return r.reshape(BH, D)

        y2 = d00 + shift_n(d01) + shift_m(d10) + shift_m(shift_n(d11))
        y2 = y2 + bn2b_ref[...]

        y2b = y2.reshape(Bblk, HW, D)
        pooled = jnp.mean(y2b, axis=1)
        h = jnp.maximum(dot(pooled, wfc1_ref[...]), 0.0)
        s = jax.nn.sigmoid(dot(h, wfc2_ref[...]))

        out = y2b * s[:, None, :] + short.reshape(Bblk, HW, D)
        o_ref[...] = out.reshape(BH, D)

    return _body


def _bottleneck(xpl, wt, alpha_t, w2, bn2b, wsc, bnscb, wfc1, wfc2,
                *, Bblk, Hh, Wh):
    G = xpl.shape[0]
    L = xpl.shape[-1]
    D = w2.shape[-1]
    Dmid = wfc1.shape[-1]
    BH = Bblk * Hh * Wh
    body = _make_block_kernel(Bblk, Hh, Wh, D, L)

    def const(shape):
        return pl.BlockSpec(shape, lambda g: (0,) * len(shape))

    return pl.pallas_call(
        body,
        out_shape=jax.ShapeDtypeStruct((G, BH, D), jnp.float32),
        grid=(G,),
        in_specs=[
            pl.BlockSpec((None, Bblk * 2 * (Hh + 1), L),
                         lambda g: (g, 0, 0)),
            const((6, L, Wh * D)),
            const((1, Wh * D)),
            const((9, D, D)),
            const((1, D)),
            const((L, Wh * D)),
            const((1, D)),
            const((D, Dmid)),
            const((Dmid, D)),
        ],
        out_specs=pl.BlockSpec((None, BH, D), lambda g: (g, 0, 0)),
        compiler_params=pltpu.CompilerParams(dimension_semantics=("parallel",)),
    )(xpl, wt, alpha_t, w2, bn2b, wsc, bnscb, wfc1, wfc2)


def _make_head_kernel(paired):
    f32 = jnp.float32

    if paired:
        def _body(a_ref, b_ref, wo_ref, wa_ref, wb_ref, o_ref, acca_ref, accb_ref):
            k = pl.program_id(1)

            @pl.when(k == 0)
            def _():
                acca_ref[...] = jnp.zeros_like(acca_ref)
                accb_ref[...] = jnp.zeros_like(accb_ref)

            acca_ref[...] += jnp.dot(a_ref[...], wo_ref[...],
                                     preferred_element_type=f32)
            accb_ref[...] += jnp.dot(b_ref[...], wo_ref[...],
                                     preferred_element_type=f32)

            @pl.when(k == pl.num_programs(1) - 1)
            def _():
                ha = jnp.maximum(jnp.dot(acca_ref[...], wa_ref[...],
                                         preferred_element_type=f32), 0.0)
                hb = jnp.maximum(jnp.dot(accb_ref[...], wa_ref[...],
                                         preferred_element_type=f32), 0.0)
                s = ha + hb
                z = jnp.dot(s, wb_ref[...], preferred_element_type=f32)
                nrm2 = jnp.sum(z * z, axis=1, keepdims=True)
                o_ref[...] = z * lax.rsqrt(jnp.maximum(nrm2, 1e-24))
        return _body

    def _body1(a_ref, wo_ref, wa_ref, wb_ref, o_ref, acca_ref):
        k = pl.program_id(1)

        @pl.when(k == 0)
        def _():
            acca_ref[...] = jnp.zeros_like(acca_ref)

        acca_ref[...] += jnp.dot(a_ref[...], wo_ref[...],
                                 preferred_element_type=f32)

        @pl.when(k == pl.num_programs(1) - 1)
        def _():
            h = jnp.maximum(jnp.dot(acca_ref[...], wa_ref[...],
                                    preferred_element_type=f32), 0.0)
            z = jnp.dot(h, wb_ref[...], preferred_element_type=f32)
            nrm2 = jnp.sum(z * z, axis=1, keepdims=True)
            o_ref[...] = z * lax.rsqrt(jnp.maximum(nrm2, 1e-24))
    return _body1


def _head(feat_flat, w_out, w_a, w_b, *, paired, tk=4096):
    B, F = feat_flat.shape
    E = w_out.shape[-1]
    if F % tk:
        tk = F
    nk = F // tk
    Bout = B // 2 if paired else B
    S = 2 if Bout % 2 == 0 else 1
    R = Bout // S

    in_specs = [pl.BlockSpec((R, tk), lambda c, k: (c, k))]
    scratch = [pltpu.VMEM((R, E), jnp.float32)]
    if paired:
        in_specs.append(pl.BlockSpec((R, tk), lambda c, k: (c + S, k)))
        scratch.append(pltpu.VMEM((R, E), jnp.float32))
    in_specs += [
        pl.BlockSpec((tk, E), lambda c, k: (k, 0)),
        pl.BlockSpec((E, E), lambda c, k: (0, 0)),
        pl.BlockSpec((E, E), lambda c, k: (0, 0)),
    ]
    operands = ([feat_flat, feat_flat] if paired else [feat_flat])
    operands += [w_out, w_a, w_b]

    return pl.pallas_call(
        _make_head_kernel(paired),
        out_shape=jax.ShapeDtypeStruct((Bout, E), jnp.float32),
        grid=(S, nk),
        in_specs=in_specs,
        out_specs=pl.BlockSpec((R, E), lambda c, k: (c, 0)),
        scratch_shapes=scratch,
        compiler_params=pltpu.CompilerParams(
            dimension_semantics=("parallel", "arbitrary")),
    )(*operands)


def _pick_bblk(B):
    for cand in (4, 2):
        if B % cand == 0 and B // cand >= 2:
            return cand
    return 1


def _toeplitz(band, off, Wh, L, D):
    s = band.shape[0]
    eye = jnp.eye(Wh, dtype=band.dtype)
    t = eye[:, None, :, None] * band[None, :, None, :]
    t = t.reshape(Wh * s, Wh * D)
    t = jnp.pad(t, ((off, max(0, L - Wh * s - off)), (0, 0)))
    return t[:L]


def kernel(x_nchw, w_conv1, w_conv2, w_sc, w_fc1, w_fc2, prelu_alpha,
           bn1_scale, bn1_shift, bn2_scale, bn2_shift, bnsc_scale, bnsc_shift,
           w_out, w_drl1, w_drl2):
    Cin = bn1_scale.shape[0]
    D = bn2_scale.shape[0]
    teacher = x_nchw.shape[1] == 2 * Cin

    if teacher:
        x_nchw = x_nchw[:, Cin:, :, :]
        x_nchw = jnp.concatenate([x_nchw, x_nchw[:, :, :, ::-1]], axis=0)
    x = jnp.transpose(x_nchw, (0, 2, 3, 1)).astype(jnp.float32)
    B, H, W, _ = x.shape
    Hh, Wh = H // 2, W // 2
    HW = Hh * Wh
    Bblk = _pick_bblk(B)
    G = B // Bblk
    BH = Bblk * HW
    L = (W + 2) * Cin
    s2 = 2 * Cin

    xbn = (x * bn1_scale + bn1_shift).astype(jnp.bfloat16)
    xp = jnp.pad(xbn, ((0, 0), (1, 1), (1, 1), (0, 0)))
    xpl = xp.reshape(B, H + 2, L).reshape(G, Bblk * (H + 2), L)

    w1r = w_conv1.reshape(3, 3 * Cin, D)
    wt = []
    for dy in range(3):
        wdp = jnp.pad(w1r[dy], ((0, 2 * s2 - 3 * Cin), (0, 0)))
        for rx in range(2):
            ta = _toeplitz(wdp[:s2], Cin * rx, Wh, L, D)
            tb = _toeplitz(wdp[s2:], Cin * rx + s2, Wh, L, D)
            wt.append(ta + tb)
    wt = jnp.stack(wt).astype(jnp.bfloat16)

    wsc_eff = (w_sc * bnsc_scale[None, :]) / bn1_scale[:, None]
    bnscb_eff = bnsc_shift - bn1_shift @ wsc_eff
    wsc_band = jnp.pad(wsc_eff, ((0, s2 - Cin), (0, 0)))
    wsc = _toeplitz(wsc_band, Cin, Wh, L, D).astype(jnp.bfloat16)

    w2 = (w_conv2 * bn2_scale[None, None, None, :]).reshape(9, D, D)
    alpha_t = jnp.tile(prelu_alpha.reshape(1, D), (1, Wh))
    bn2b = bn2_shift.reshape(1, D)
    bnscb = bnscb_eff.reshape(1, D)

    feat = _bottleneck(xpl, wt, alpha_t, w2, bn2b, wsc, bnscb,
                       w_fc1, w_fc2, Bblk=Bblk, Hh=Hh, Wh=Wh)
    feat_flat = feat.reshape(B, HW * D)

    w_out_p = w_out.reshape(Hh, Wh, D, -1).swapaxes(0, 1).reshape(w_out.shape)

    return _head(feat_flat, w_out_p, w_drl1, w_drl2, paired=teacher)


# speedup vs baseline: 5.3907x; 1.0295x over previous
import jax
import jax.numpy as jnp
from jax import lax
from jax.experimental import pallas as pl
from jax.experimental.pallas import tpu as pltpu


def _make_block_kernel(Bblk, Hh, Wh, D, L):
    HW = Hh * Wh
    BH = Bblk * HW
    BHh = Bblk * Hh
    f32 = jnp.float32

    def _body(xpl_ref, wt_ref, alpha_ref, w2_ref, bn2b_ref,
              wsc_ref, bnscb_ref, wfc1_ref, wfc2_ref, o_ref):
        def dot(a, b):
            return jnp.dot(a, b, preferred_element_type=f32)

        xv = xpl_ref[...].reshape(Bblk, Hh + 1, 2, L)
        xe = xv[:, :, 0, :]
        xo = xv[:, :, 1, :]

        def rows(src, off):
            return src[:, off:off + Hh, :].reshape(BHh, L)

        def phase(ry, rx):
            if ry == 0:
                srcs = (rows(xe, 0), rows(xo, 0), rows(xe, 1))
            else:
                srcs = (rows(xo, 0), rows(xe, 1), rows(xo, 1))
            y = (dot(srcs[0], wt_ref[0 * 2 + rx])
                 + dot(srcs[1], wt_ref[1 * 2 + rx])
                 + dot(srcs[2], wt_ref[2 * 2 + rx]))
            return jnp.where(y >= 0.0, y, y * alpha_ref[...])

        y00 = phase(0, 0)
        y01 = phase(0, 1)
        y10 = phase(1, 0)
        y11 = phase(1, 1)

        shf = dot(rows(xo, 0), wsc_ref[...])

        def to_rows(yf):
            pieces = [yf[j * Hh:(j + 1) * Hh, n * D:(n + 1) * D]
                      for j in range(Bblk) for n in range(Wh)]
            return jnp.concatenate(pieces, axis=0)

        ph0 = to_rows(y00)
        ph1 = to_rows(y01)
        ph2 = to_rows(y10)
        ph3 = to_rows(y11)
        short = to_rows(shf) + bnscb_ref[...]

        d00 = (dot(ph0, w2_ref[4]) + dot(ph1, w2_ref[5])
               + dot(ph2, w2_ref[7]) + dot(ph3, w2_ref[8]))
        d01 = dot(ph1, w2_ref[3]) + dot(ph3, w2_ref[6])
        d10 = dot(ph2, w2_ref[1]) + dot(ph3, w2_ref[2])
        d11 = dot(ph3, w2_ref[0])

        def shift_n(v):
            r = v.reshape(Bblk, Wh, Hh * D)
            r = jnp.concatenate(
                [jnp.zeros((Bblk, 1, Hh * D), f32), r[:, :Wh - 1, :]], axis=1)
            return r.reshape(BH, D)

        def shift_m(v):
            r = v.reshape(Bblk * Wh, Hh, D)
            r = jnp.concatenate(
                [jnp.zeros((Bblk * Wh, 1, D), f32), r[:, :Hh - 1, :]], axis=1)
            return r.reshape(BH, D)

        y2 = d00 + shift_n(d01) + shift_m(d10) + shift_m(shift_n(d11))
        y2 = y2 + bn2b_ref[...]

        y2b = y2.reshape(Bblk, HW, D)
        pooled = jnp.mean(y2b, axis=1)
        h = jnp.maximum(dot(pooled, wfc1_ref[...]), 0.0)
        s = jax.nn.sigmoid(dot(h, wfc2_ref[...]))

        out = y2b * s[:, None, :] + short.reshape(Bblk, HW, D)
        o_ref[...] = out.reshape(BH, D)

    return _body


def _bottleneck(xpl, wt, alpha_t, w2, bn2b, wsc, bnscb, wfc1, wfc2,
                *, Bblk, Hh, Wh):
    G = xpl.shape[0]
    L = xpl.shape[-1]
    D = w2.shape[-1]
    Dmid = wfc1.shape[-1]
    BH = Bblk * Hh * Wh
    body = _make_block_kernel(Bblk, Hh, Wh, D, L)

    def const(shape):
        return pl.BlockSpec(shape, lambda g: (0,) * len(shape))

    return pl.pallas_call(
        body,
        out_shape=jax.ShapeDtypeStruct((G, BH, D), jnp.float32),
        grid=(G,),
        in_specs=[
            pl.BlockSpec((None, Bblk * 2 * (Hh + 1), L),
                         lambda g: (g, 0, 0)),
            const((6, L, Wh * D)),
            const((1, Wh * D)),
            const((9, D, D)),
            const((1, D)),
            const((L, Wh * D)),
            const((1, D)),
            const((D, Dmid)),
            const((Dmid, D)),
        ],
        out_specs=pl.BlockSpec((None, BH, D), lambda g: (g, 0, 0)),
        compiler_params=pltpu.CompilerParams(dimension_semantics=("parallel",)),
    )(xpl, wt, alpha_t, w2, bn2b, wsc, bnscb, wfc1, wfc2)


def _make_head_kernel(paired):
    f32 = jnp.float32

    if paired:
        def _body(a_ref, b_ref, wo_ref, wa_ref, wb_ref, o_ref, acca_ref, accb_ref):
            k = pl.program_id(1)

            @pl.when(k == 0)
            def _():
                acca_ref[...] = jnp.zeros_like(acca_ref)
                accb_ref[...] = jnp.zeros_like(accb_ref)

            acca_ref[...] += jnp.dot(a_ref[...], wo_ref[...],
                                     preferred_element_type=f32)
            accb_ref[...] += jnp.dot(b_ref[...], wo_ref[...],
                                     preferred_element_type=f32)

            @pl.when(k == pl.num_programs(1) - 1)
            def _():
                ha = jnp.maximum(jnp.dot(acca_ref[...], wa_ref[...],
                                         preferred_element_type=f32), 0.0)
                hb = jnp.maximum(jnp.dot(accb_ref[...], wa_ref[...],
                                         preferred_element_type=f32), 0.0)
                s = ha + hb
                z = jnp.dot(s, wb_ref[...], preferred_element_type=f32)
                nrm2 = jnp.sum(z * z, axis=1, keepdims=True)
                o_ref[...] = z * lax.rsqrt(jnp.maximum(nrm2, 1e-24))
        return _body

    def _body1(a_ref, wo_ref, wa_ref, wb_ref, o_ref, acca_ref):
        k = pl.program_id(1)

        @pl.when(k == 0)
        def _():
            acca_ref[...] = jnp.zeros_like(acca_ref)

        acca_ref[...] += jnp.dot(a_ref[...], wo_ref[...],
                                 preferred_element_type=f32)

        @pl.when(k == pl.num_programs(1) - 1)
        def _():
            h = jnp.maximum(jnp.dot(acca_ref[...], wa_ref[...],
                                    preferred_element_type=f32), 0.0)
            z = jnp.dot(h, wb_ref[...], preferred_element_type=f32)
            nrm2 = jnp.sum(z * z, axis=1, keepdims=True)
            o_ref[...] = z * lax.rsqrt(jnp.maximum(nrm2, 1e-24))
    return _body1


def _head(feat_flat, w_out, w_a, w_b, *, paired, tk=4096):
    B, F = feat_flat.shape
    E = w_out.shape[-1]
    if F % tk:
        tk = F
    nk = F // tk
    Bout = B // 2 if paired else B
    S = 2 if Bout % 2 == 0 else 1
    R = Bout // S

    in_specs = [pl.BlockSpec((R, tk), lambda c, k: (c, k))]
    scratch = [pltpu.VMEM((R, E), jnp.float32)]
    if paired:
        in_specs.append(pl.BlockSpec((R, tk), lambda c, k: (c + S, k)))
        scratch.append(pltpu.VMEM((R, E), jnp.float32))
    in_specs += [
        pl.BlockSpec((tk, E), lambda c, k: (k, 0)),
        pl.BlockSpec((E, E), lambda c, k: (0, 0)),
        pl.BlockSpec((E, E), lambda c, k: (0, 0)),
    ]
    operands = ([feat_flat, feat_flat] if paired else [feat_flat])
    operands += [w_out, w_a, w_b]

    return pl.pallas_call(
        _make_head_kernel(paired),
        out_shape=jax.ShapeDtypeStruct((Bout, E), jnp.float32),
        grid=(S, nk),
        in_specs=in_specs,
        out_specs=pl.BlockSpec((R, E), lambda c, k: (c, 0)),
        scratch_shapes=scratch,
        compiler_params=pltpu.CompilerParams(
            dimension_semantics=("parallel", "arbitrary")),
    )(*operands)


def _pick_bblk(B):
    for cand in (8, 4, 2):
        if B % cand == 0 and B // cand >= 2:
            return cand
    return 1


def _toeplitz(band, off, Wh, L, D):
    s = band.shape[0]
    eye = jnp.eye(Wh, dtype=band.dtype)
    t = eye[:, None, :, None] * band[None, :, None, :]
    t = t.reshape(Wh * s, Wh * D)
    t = jnp.pad(t, ((off, max(0, L - Wh * s - off)), (0, 0)))
    return t[:L]


def kernel(x_nchw, w_conv1, w_conv2, w_sc, w_fc1, w_fc2, prelu_alpha,
           bn1_scale, bn1_shift, bn2_scale, bn2_shift, bnsc_scale, bnsc_shift,
           w_out, w_drl1, w_drl2):
    Cin = bn1_scale.shape[0]
    D = bn2_scale.shape[0]
    teacher = x_nchw.shape[1] == 2 * Cin

    if teacher:
        x_nchw = x_nchw[:, Cin:, :, :]
        x_nchw = jnp.concatenate([x_nchw, x_nchw[:, :, :, ::-1]], axis=0)
    x = jnp.transpose(x_nchw, (0, 2, 3, 1)).astype(jnp.float32)
    B, H, W, _ = x.shape
    Hh, Wh = H // 2, W // 2
    HW = Hh * Wh
    Bblk = _pick_bblk(B)
    G = B // Bblk
    BH = Bblk * HW
    L = (W + 2) * Cin
    s2 = 2 * Cin

    xbn = (x * bn1_scale + bn1_shift).astype(jnp.bfloat16)
    xp = jnp.pad(xbn, ((0, 0), (1, 1), (1, 1), (0, 0)))
    xpl = xp.reshape(B, H + 2, L).reshape(G, Bblk * (H + 2), L)

    w1r = w_conv1.reshape(3, 3 * Cin, D)
    wt = []
    for dy in range(3):
        wdp = jnp.pad(w1r[dy], ((0, 2 * s2 - 3 * Cin), (0, 0)))
        for rx in range(2):
            ta = _toeplitz(wdp[:s2], Cin * rx, Wh, L, D)
            tb = _toeplitz(wdp[s2:], Cin * rx + s2, Wh, L, D)
            wt.append(ta + tb)
    wt = jnp.stack(wt).astype(jnp.bfloat16)

    wsc_eff = (w_sc * bnsc_scale[None, :]) / bn1_scale[:, None]
    bnscb_eff = bnsc_shift - bn1_shift @ wsc_eff
    wsc_band = jnp.pad(wsc_eff, ((0, s2 - Cin), (0, 0)))
    wsc = _toeplitz(wsc_band, Cin, Wh, L, D).astype(jnp.bfloat16)

    w2 = (w_conv2 * bn2_scale[None, None, None, :]).reshape(9, D, D)
    alpha_t = jnp.tile(prelu_alpha.reshape(1, D), (1, Wh))
    bn2b = bn2_shift.reshape(1, D)
    bnscb = bnscb_eff.reshape(1, D)

    feat = _bottleneck(xpl, wt, alpha_t, w2, bn2b, wsc, bnscb,
                       w_fc1, w_fc2, Bblk=Bblk, Hh=Hh, Wh=Wh)
    feat_flat = feat.reshape(B, HW * D)

    w_out_p = w_out.reshape(Hh, Wh, D, -1).swapaxes(0, 1).reshape(w_out.shape)

    return _head(feat_flat, w_out_p, w_drl1, w_drl2, paired=teacher)
